# Initial kernel scaffold; baseline (speedup 1.0000x reference)
import jax
import jax.numpy as jnp
from jax.experimental import pallas as pl


def kernel(xyz, features, W0, b0, g0, be0, W1, b1, g1, be1, W2, b2, g2, be2):
    raise NotImplementedError("write your pallas kernel here")


# speedup vs baseline: 2.2673x; 2.2673x over previous
import functools

import jax
import jax.numpy as jnp
from jax.experimental import pallas as pl
from jax.experimental.pallas import tpu as pltpu

B = 8
N = 4096
NPOINT = 1024
NSAMPLE = 32
CIN = 128
EPS = 1e-5
BIGF = 1e10
CT = 128
TM = 2048
PN = B * NPOINT * NSAMPLE


def _fps_body(xyz_ref, idx_ref, nxyz_ref):
    xs = xyz_ref[0]
    ys = xyz_ref[1]
    zs = xyz_ref[2]
    iota = jax.lax.broadcasted_iota(jnp.int32, (B, N), 1)
    row_iota = jax.lax.broadcasted_iota(jnp.int32, (B, NPOINT), 0)
    iota_np = jax.lax.broadcasted_iota(jnp.int32, (B, NPOINT), 1)

    def body(i, carry):
        dist, far, oidx, ox, oy, oz = carry
        oh = iota == far
        cx = jnp.sum(jnp.where(oh, xs, 0.0), axis=1, keepdims=True)
        cy = jnp.sum(jnp.where(oh, ys, 0.0), axis=1, keepdims=True)
        cz = jnp.sum(jnp.where(oh, zs, 0.0), axis=1, keepdims=True)
        sel = (iota_np == i) & (row_iota >= 0)
        oidx = oidx + jnp.where(sel, jnp.broadcast_to(far, (B, NPOINT)), 0)
        ox = ox + jnp.where(sel, jnp.broadcast_to(cx, (B, NPOINT)), 0.0)
        oy = oy + jnp.where(sel, jnp.broadcast_to(cy, (B, NPOINT)), 0.0)
        oz = oz + jnp.where(sel, jnp.broadcast_to(cz, (B, NPOINT)), 0.0)
        d = (xs - cx) ** 2 + (ys - cy) ** 2 + (zs - cz) ** 2
        dist = jnp.minimum(dist, d)
        m = jnp.max(dist, axis=1, keepdims=True)
        far2 = jnp.min(jnp.where(dist == m, iota, N), axis=1,
                       keepdims=True).astype(jnp.int32)
        return dist, far2, oidx, ox, oy, oz

    dist0 = jnp.full((B, N), BIGF, jnp.float32)
    far0 = jnp.zeros((B, 1), jnp.int32)
    zf = jnp.zeros((B, NPOINT), jnp.float32)
    zi = jnp.zeros((B, NPOINT), jnp.int32)
    _, _, oidx, ox, oy, oz = jax.lax.fori_loop(
        0, NPOINT, body, (dist0, far0, zi, zf, zf, zf))
    idx_ref[...] = oidx
    nxyz_ref[:, 0, :] = ox
    nxyz_ref[:, 1, :] = oy
    nxyz_ref[:, 2, :] = oz


def _fps(xyz_t):
    return pl.pallas_call(
        _fps_body,
        out_shape=[
            jax.ShapeDtypeStruct((B, NPOINT), jnp.int32),
            jax.ShapeDtypeStruct((B, 3, NPOINT), jnp.float32),
        ],
    )(xyz_t)


def _knn_body(xyz_ref, nxyz_ref, idx_ref, d_scr):
    xmat = xyz_ref[0]
    cmat = nxyz_ref[0]
    mm = jnp.dot(xmat, cmat, preferred_element_type=jnp.float32)
    d = -2.0 * mm
    d = d + jnp.sum(xmat * xmat, axis=1, keepdims=True)
    d = d + jnp.sum(cmat * cmat, axis=0, keepdims=True)
    d_scr[...] = d
    iota = jax.lax.broadcasted_iota(jnp.int32, (N, CT), 0)

    def ext(k, _):
        dv = d_scr[...]
        m = jnp.min(dv, axis=0, keepdims=True)
        am = jnp.min(jnp.where(dv == m, iota, N), axis=0,
                     keepdims=True).astype(jnp.int32)
        idx_ref[0, pl.ds(k, 1), :] = am
        d_scr[...] = jnp.where(iota == am, BIGF, dv)
        return 0

    jax.lax.fori_loop(0, NSAMPLE, ext, 0)


def _knn(xyz, nxyz_t):
    return pl.pallas_call(
        _knn_body,
        grid=(B, NPOINT // CT),
        in_specs=[
            pl.BlockSpec((1, N, 3), lambda b, t: (b, 0, 0)),
            pl.BlockSpec((1, 3, CT), lambda b, t: (b, 0, t)),
        ],
        out_specs=pl.BlockSpec((1, NSAMPLE, CT), lambda b, t: (b, 0, t)),
        out_shape=jax.ShapeDtypeStruct((B, NSAMPLE, NPOINT), jnp.int32),
        scratch_shapes=[pltpu.VMEM((N, CT), jnp.float32)],
    )(xyz, nxyz_t)


def _acc_stats(y, s_ref, q_ref):
    ps = jnp.sum(y, axis=0, keepdims=True)
    pq = jnp.sum(y * y, axis=0, keepdims=True)

    @pl.when(pl.program_id(0) == 0)
    def _():
        s_ref[...] = ps
        q_ref[...] = pq

    @pl.when(pl.program_id(0) != 0)
    def _():
        s_ref[...] = s_ref[...] + ps
        q_ref[...] = q_ref[...] + pq


def _mlp0_body(gx_ref, gf_ref, wx_ref, wf_ref, b_ref, y_ref, s_ref, q_ref):
    y = jnp.dot(gf_ref[...], wf_ref[...], preferred_element_type=jnp.float32)
    y = y + jnp.dot(gx_ref[...], wx_ref[...], preferred_element_type=jnp.float32)
    y = y + b_ref[...]
    y_ref[...] = y
    _acc_stats(y, s_ref, q_ref)


def _norm_relu(y, s_ref, q_ref, g_ref, be_ref):
    mean = s_ref[...] / PN
    var = q_ref[...] / PN - mean * mean
    xn = (y - mean) / jnp.sqrt(var + EPS) * g_ref[...] + be_ref[...]
    return jnp.maximum(xn, 0.0)


def _mlp_mid_body(y0_ref, s0_ref, q0_ref, g_ref, be_ref, w_ref, b_ref,
                  y_ref, s_ref, q_ref):
    x = _norm_relu(y0_ref[...], s0_ref, q0_ref, g_ref, be_ref)
    y = jnp.dot(x, w_ref[...], preferred_element_type=jnp.float32) + b_ref[...]
    y_ref[...] = y
    _acc_stats(y, s_ref, q_ref)


def _mlp_out_body(y2_ref, s2_ref, q2_ref, g_ref, be_ref, o_ref):
    x = _norm_relu(y2_ref[...], s2_ref, q2_ref, g_ref, be_ref)
    xr = x.reshape(TM // NSAMPLE, NSAMPLE, x.shape[-1])
    o_ref[...] = jnp.max(xr, axis=1)


def _row_spec(c):
    return pl.BlockSpec((TM, c), lambda s: (s, 0))


def _full_spec(shape):
    return pl.BlockSpec(shape, lambda s: tuple(0 for _ in shape))


def _stat_specs():
    return [pl.BlockSpec((1, s), lambda i: (0, 0)) for s in (0,)]


def _mlp(gx, gf, params):
    (w0, b0, g0, be0), (w1, b1, g1, be1), (w2, b2, g2, be2) = params
    steps = PN // TM
    c1, c2 = 128, 256
    w0x = jnp.transpose(w0[:, :3])
    w0f = jnp.transpose(w0[:, 3:])
    w1t = jnp.transpose(w1)
    w2t = jnp.transpose(w2)
    r = lambda v: v.reshape(1, -1)

    y0, s0, q0 = pl.pallas_call(
        _mlp0_body,
        grid=(steps,),
        in_specs=[
            _row_spec(3), _row_spec(CIN),
            _full_spec((3, c1)), _full_spec((CIN, c1)), _full_spec((1, c1)),
        ],
        out_specs=[
            _row_spec(c1),
            pl.BlockSpec((1, c1), lambda s: (0, 0)),
            pl.BlockSpec((1, c1), lambda s: (0, 0)),
        ],
        out_shape=[
            jax.ShapeDtypeStruct((PN, c1), jnp.float32),
            jax.ShapeDtypeStruct((1, c1), jnp.float32),
            jax.ShapeDtypeStruct((1, c1), jnp.float32),
        ],
    )(gx, gf, w0x, w0f, r(b0))

    def mid(y, s, q, g, be, wt, b, cout):
        return pl.pallas_call(
            _mlp_mid_body,
            grid=(steps,),
            in_specs=[
                _row_spec(y.shape[-1]),
                _full_spec((1, y.shape[-1])), _full_spec((1, y.shape[-1])),
                _full_spec((1, y.shape[-1])), _full_spec((1, y.shape[-1])),
                _full_spec((y.shape[-1], cout)), _full_spec((1, cout)),
            ],
            out_specs=[
                _row_spec(cout),
                pl.BlockSpec((1, cout), lambda s: (0, 0)),
                pl.BlockSpec((1, cout), lambda s: (0, 0)),
            ],
            out_shape=[
                jax.ShapeDtypeStruct((PN, cout), jnp.float32),
                jax.ShapeDtypeStruct((1, cout), jnp.float32),
                jax.ShapeDtypeStruct((1, cout), jnp.float32),
            ],
        )(y, s, q, r(g), r(be), wt, b)

    y1, s1, q1 = mid(y0, s0, q0, g0, be0, w1t, r(b1), c1)
    y2, s2, q2 = mid(y1, s1, q1, g1, be1, w2t, r(b2), c2)

    out = pl.pallas_call(
        _mlp_out_body,
        grid=(steps,),
        in_specs=[
            _row_spec(c2),
            _full_spec((1, c2)), _full_spec((1, c2)),
            _full_spec((1, c2)), _full_spec((1, c2)),
        ],
        out_specs=pl.BlockSpec((TM // NSAMPLE, c2), lambda s: (s, 0)),
        out_shape=jax.ShapeDtypeStruct((B * NPOINT, c2), jnp.float32),
    )(y2, s2, q2, r(g2), r(be2))
    return out


def kernel(xyz, features, W0, b0, g0, be0, W1, b1, g1, be1, W2, b2, g2, be2):
    xyz_t = jnp.transpose(xyz, (2, 0, 1))
    _, nxyz_b = _fps(xyz_t)
    idx_t = _knn(xyz, nxyz_b)
    idx = jnp.transpose(idx_t, (0, 2, 1))
    new_xyz = jnp.transpose(nxyz_b, (0, 2, 1))

    feat_t = jnp.swapaxes(features, 1, 2)
    idx_flat = idx.reshape(B, NPOINT * NSAMPLE)
    gf = jnp.take_along_axis(feat_t, idx_flat[..., None], axis=1)
    gxyz = jnp.take_along_axis(xyz, idx_flat[..., None], axis=1)
    gxyz = gxyz.reshape(B, NPOINT, NSAMPLE, 3) - new_xyz[:, :, None, :]

    gx = gxyz.reshape(PN, 3)
    gf = gf.reshape(PN, CIN)
    params = [(W0, b0, g0, be0), (W1, b1, g1, be1), (W2, b2, g2, be2)]
    outf = _mlp(gx, gf, params)
    new_features = jnp.transpose(outf.reshape(B, NPOINT, 256), (0, 2, 1))
    return new_xyz, new_features
